# baseline (speedup 1.0000x reference)
import functools
import math

import jax
import jax.numpy as jnp
from jax import lax
from jax.experimental import pallas as pl
from jax.experimental.pallas import tpu as pltpu
from jax.experimental.pallas import tpu_sc as plsc

INTERPRET = False
BN_EPS = 1e-05
NW = 32
EC = 128
ROW_BLK = 1000


def _mm_relu_body(x_ref, w_ref, b_ref, o_ref):
    o_ref[...] = jax.nn.relu(
        jnp.dot(x_ref[...], w_ref[...], preferred_element_type=jnp.float32)
        + b_ref[...][None, :]
    )


def _mm_body(x_ref, w_ref, b_ref, o_ref):
    o_ref[...] = (
        jnp.dot(x_ref[...], w_ref[...], preferred_element_type=jnp.float32)
        + b_ref[...][None, :]
    )


def _matmul(x, w, b, relu):
    n, k = x.shape
    m = w.shape[1]
    body = _mm_relu_body if relu else _mm_body
    return pl.pallas_call(
        body,
        grid=(n // ROW_BLK,),
        in_specs=[
            pl.BlockSpec((ROW_BLK, k), lambda i: (i, 0)),
            pl.BlockSpec((k, m), lambda i: (0, 0)),
            pl.BlockSpec((m,), lambda i: (0,)),
        ],
        out_specs=pl.BlockSpec((ROW_BLK, m), lambda i: (i, 0)),
        out_shape=jax.ShapeDtypeStruct((n, m), jnp.float32),
        interpret=INTERPRET,
    )(x, w, b)


def _spmm_sc_call(n, d, rpw, h, cols, vals, rows, bounds, scale, beta):
    nwr = NW * rpw
    mesh = plsc.VectorSubcoreMesh(core_axis_name="c", subcore_axis_name="s",
                                  num_cores=2, num_subcores=16)

    @functools.partial(
        pl.kernel,
        out_type=jax.ShapeDtypeStruct((nwr, d), jnp.float32),
        mesh=mesh,
        scratch_types=[
            pltpu.VMEM((EC,), jnp.int32),
            pltpu.VMEM((EC,), jnp.int32),
            pltpu.VMEM((EC,), jnp.float32),
            pltpu.VMEM((EC, d), jnp.float32),
            pltpu.VMEM((rpw, d), jnp.float32),
            pltpu.VMEM((2 * NW + 16,), jnp.int32),
            pltpu.VMEM((d,), jnp.float32),
            pltpu.VMEM((d,), jnp.float32),
            pltpu.SemaphoreType.DMA,
        ],
        compiler_params=pltpu.CompilerParams(use_tc_tiling_on_sc=False),
        interpret=INTERPRET,
    )
    def spmm(h_hbm, cols_hbm, vals_hbm, rows_hbm, bounds_hbm, scale_hbm,
             beta_hbm, out_hbm, cols_v, rows_v, vals_v, gat_v, acc_v,
             bnd_v, sc_v, bt_v, sem):
        wid = lax.axis_index("s") * 2 + lax.axis_index("c")
        r0 = pl.multiple_of(wid * rpw, 8)

        pltpu.sync_copy(bounds_hbm, bnd_v)
        pltpu.sync_copy(scale_hbm, sc_v)
        pltpu.sync_copy(beta_hbm, bt_v)
        e_lo = bnd_v[pl.ds(wid, 16)][0]
        e_hi = bnd_v[pl.ds(NW + wid, 16)][0]

        zeros = jnp.zeros((16,), jnp.float32)

        def zero_body(i, _):
            for k in range(d // 16):
                acc_v[i, pl.ds(k * 16, 16)] = zeros
            return 0

        lax.fori_loop(0, rpw, zero_body, 0)

        def chunk_body(ci, _):
            base = pl.multiple_of(e_lo + ci * EC, 8)
            pltpu.sync_copy(cols_hbm.at[pl.ds(base, EC)], cols_v)
            pltpu.sync_copy(rows_hbm.at[pl.ds(base, EC)], rows_v)
            pltpu.sync_copy(vals_hbm.at[pl.ds(base, EC)], vals_v)
            pltpu.async_copy(h_hbm.at[cols_v], gat_v, sem).wait()

            def group_body(g, _):
                gb = g * 16
                rows16 = rows_v[pl.ds(gb, 16)]
                vals16 = vals_v[pl.ds(gb, 16)]
                lrow16 = rows16 - r0
                ok16 = (lrow16 >= 0) & (lrow16 < rpw)
                lrow16 = jnp.where(ok16, lrow16, 0)
                vals16 = jnp.where(ok16, vals16, 0.0)
                for j in range(16):
                    lrow = lrow16[j]
                    vv = jnp.full((16,), vals16[j], jnp.float32)
                    for k in range(d // 16):
                        sl = pl.ds(k * 16, 16)
                        acc_v[lrow, sl] = acc_v[lrow, sl] + gat_v[gb + j, sl] * vv
                return 0

            lax.fori_loop(0, EC // 16, group_body, 0)
            return 0

        nch = lax.div(e_hi - e_lo + (EC - 1), EC)
        lax.fori_loop(0, nch, chunk_body, 0)

        def affine_body(i, _):
            for k in range(d // 16):
                sl = pl.ds(k * 16, 16)
                acc_v[i, sl] = acc_v[i, sl] * sc_v[sl] + bt_v[sl]
            return 0

        lax.fori_loop(0, rpw, affine_body, 0)
        pltpu.sync_copy(acc_v, out_hbm.at[pl.ds(r0, rpw)])

    out = spmm(h, cols, vals, rows, bounds, scale, beta)
    return out[:n]


def _prep_edges(rows, n, rpw):
    e = rows.shape[0]
    epad = (-(-e // EC) + 1) * EC
    bnd = jnp.arange(1, NW, dtype=jnp.int32) * rpw
    s = jnp.searchsorted(rows, bnd).astype(jnp.int32)
    estart = jnp.concatenate([jnp.zeros((1,), jnp.int32), (s // 8) * 8])
    e_end = jnp.int32(-(-e // 8) * 8)
    eend = jnp.concatenate([-(-s // 8) * 8, e_end[None]])
    return epad, jnp.concatenate([estart, eend, jnp.zeros((16,), jnp.int32)])


def _pad_edges(cols, vals, rows, n, epad):
    e = cols.shape[0]
    pad = epad - e
    cols = jnp.pad(cols.astype(jnp.int32), (0, pad))
    rows = jnp.pad(rows.astype(jnp.int32), (0, pad), constant_values=n)
    vals = jnp.pad(vals, (0, pad))
    return cols, vals, rows


def kernel(x, edge_index, W_fe, b_fe, bn_gamma, bn_beta, W_fp, b_fp,
           adj1_row, adj1_col, adj1_val, adj2_row, adj2_col, adj2_val):
    n = x.shape[0]
    rpw = ((-(-n // NW)) + 7) // 8 * 8
    h = _matmul(x, W_fe, b_fe, relu=True)

    s = 1.0 / math.sqrt(1.0 + BN_EPS)
    g1, g2 = bn_gamma[:64] * s, bn_gamma[64:] * s
    bt1, bt2 = bn_beta[:64], bn_beta[64:]
    ones = jnp.ones((128,), jnp.float32)
    zeros = jnp.zeros((128,), jnp.float32)

    ep1, bounds1 = _prep_edges(adj1_row, n, rpw)
    c1, v1, r1 = _pad_edges(adj1_col, adj1_val, adj1_row, n, ep1)
    ep2, bounds2 = _prep_edges(adj2_row, n, rpw)
    c2, v2, r2 = _pad_edges(adj2_col, adj2_val, adj2_row, n, ep2)

    h1a = _spmm_sc_call(n, 64, rpw, h, c1, v1, r1, bounds1, g1, bt1)
    h1b = _spmm_sc_call(n, 64, rpw, h, c2, v2, r2, bounds2, g2, bt2)
    h1 = jnp.concatenate([h1a, h1b], axis=1)

    h2a = _spmm_sc_call(n, 128, rpw, h1, c1, v1, r1, bounds1, ones, zeros)
    h2b = _spmm_sc_call(n, 128, rpw, h1, c2, v2, r2, bounds2, ones, zeros)
    h2 = jnp.concatenate([h2a, h2b], axis=1)

    xcat = jnp.concatenate([h, h1, h2], axis=1)
    return _matmul(xcat, W_fp, b_fp, relu=False)


# speedup vs baseline: 2.6963x; 2.6963x over previous
import functools

import jax
import jax.numpy as jnp
from jax import lax
from jax.experimental import pallas as pl
from jax.experimental.pallas import tpu as pltpu
from jax.experimental.pallas import tpu_sc as plsc

INTERPRET = False
BN_EPS = 1e-05
NW = 32
NSC = 2
NT = 16
EC = 128
NWR = 10240
RPT = NWR // NT
ROW_BLK = 1024


def _mm_relu_scale_body(x_ref, w_ref, b_ref, d1_ref, d2_ref,
                        h_ref, hs1_ref, hs2_ref):
    i = pl.program_id(0)
    d1 = d1_ref[pl.ds(i * ROW_BLK, ROW_BLK)]
    d2 = d2_ref[pl.ds(i * ROW_BLK, ROW_BLK)]
    h = jax.nn.relu(
        jnp.dot(x_ref[...], w_ref[...], preferred_element_type=jnp.float32)
        + b_ref[...][None, :])
    h_ref[...] = h
    hs1_ref[...] = h * d1[:, None]
    hs2_ref[...] = h * d2[:, None]


def _encoder(x, w, b, d1, d2):
    n, k = x.shape
    m = w.shape[1]
    return pl.pallas_call(
        _mm_relu_scale_body,
        grid=(n // ROW_BLK,),
        in_specs=[
            pl.BlockSpec((ROW_BLK, k), lambda i: (i, 0)),
            pl.BlockSpec((k, m), lambda i: (0, 0)),
            pl.BlockSpec((m,), lambda i: (0,)),
            pl.BlockSpec((n,), lambda i: (0,)),
            pl.BlockSpec((n,), lambda i: (0,)),
        ],
        out_specs=[pl.BlockSpec((ROW_BLK, m), lambda i: (i, 0))] * 3,
        out_shape=[jax.ShapeDtypeStruct((n, m), jnp.float32)] * 3,
        interpret=INTERPRET,
    )(x, w, b, d1, d2)


def _h1_body(a0_ref, a1_ref, b0_ref, b1_ref, d1_ref, d2_ref, g_ref, bt_ref,
             h1_ref, hs1_ref, hs2_ref):
    s = 1.0 / jnp.sqrt(1.0 + BN_EPS)
    i = pl.program_id(0)
    d1 = d1_ref[pl.ds(i * ROW_BLK, ROW_BLK)]
    d2 = d2_ref[pl.ds(i * ROW_BLK, ROW_BLK)]
    ha = (a0_ref[...] + a1_ref[...]) * d1[:, None]
    hb = (b0_ref[...] + b1_ref[...]) * d2[:, None]
    h1 = jnp.concatenate([ha, hb], axis=1) * (g_ref[...][None, :] * s) \
        + bt_ref[...][None, :]
    h1_ref[...] = h1
    hs1_ref[...] = h1 * d1[:, None]
    hs2_ref[...] = h1 * d2[:, None]


def _h1_producer(a0, a1, b0, b1, d1, d2, gamma, beta, n):
    m = a0.shape[1]
    return pl.pallas_call(
        _h1_body,
        grid=(n // ROW_BLK,),
        in_specs=[pl.BlockSpec((ROW_BLK, m), lambda i: (i, 0))] * 4
        + [pl.BlockSpec((n,), lambda i: (0,))] * 2
        + [pl.BlockSpec((2 * m,), lambda i: (0,))] * 2,
        out_specs=[pl.BlockSpec((ROW_BLK, 2 * m), lambda i: (i, 0))] * 3,
        out_shape=[jax.ShapeDtypeStruct((n, 2 * m), jnp.float32)] * 3,
        interpret=INTERPRET,
    )(a0, a1, b0, b1, d1, d2, gamma, beta)


def _final_body(h_ref, h1_ref, a0_ref, a1_ref, b0_ref, b1_ref,
                d1_ref, d2_ref, w_ref, bias_ref, o_ref):
    i = pl.program_id(0)
    d1 = d1_ref[pl.ds(i * ROW_BLK, ROW_BLK)]
    d2 = d2_ref[pl.ds(i * ROW_BLK, ROW_BLK)]
    h2a = (a0_ref[...] + a1_ref[...]) * d1[:, None]
    h2b = (b0_ref[...] + b1_ref[...]) * d2[:, None]
    xcat = jnp.concatenate([h_ref[...], h1_ref[...], h2a, h2b], axis=1)
    o_ref[...] = (
        jnp.dot(xcat, w_ref[...], preferred_element_type=jnp.float32)
        + bias_ref[...][None, :])


def _final(h, h1, a0, a1, b0, b1, d1, d2, w, bias, n):
    m = w.shape[1]
    return pl.pallas_call(
        _final_body,
        grid=(n // ROW_BLK,),
        in_specs=[
            pl.BlockSpec((ROW_BLK, 64), lambda i: (i, 0)),
            pl.BlockSpec((ROW_BLK, 128), lambda i: (i, 0)),
        ]
        + [pl.BlockSpec((ROW_BLK, 128), lambda i: (i, 0))] * 4
        + [pl.BlockSpec((n,), lambda i: (0,))] * 2
        + [
            pl.BlockSpec((448, m), lambda i: (0, 0)),
            pl.BlockSpec((m,), lambda i: (0,)),
        ],
        out_specs=pl.BlockSpec((ROW_BLK, m), lambda i: (i, 0)),
        out_shape=jax.ShapeDtypeStruct((n, m), jnp.float32),
        interpret=INTERPRET,
    )(h, h1, a0, a1, b0, b1, d1, d2, w, bias)


NBUF = 2


def _spmm_sc_call(d, k_chunks, hs, idxp):
    mesh = plsc.VectorSubcoreMesh(core_axis_name="c", subcore_axis_name="s",
                                  num_cores=2, num_subcores=16)

    @functools.partial(
        pl.kernel,
        out_type=jax.ShapeDtypeStruct((NSC, NWR, d), jnp.float32),
        mesh=mesh,
        scratch_types=[
            [pltpu.VMEM((2, EC), jnp.int32) for _ in range(NBUF)],
            [pltpu.VMEM((EC, d), jnp.float32) for _ in range(NBUF)],
            pltpu.VMEM((16, d), jnp.float32),
            pltpu.VMEM_SHARED((NWR, d), jnp.float32),
            [pltpu.SemaphoreType.DMA for _ in range(NBUF)],
            [pltpu.SemaphoreType.DMA for _ in range(NBUF)],
        ],
        compiler_params=pltpu.CompilerParams(use_tc_tiling_on_sc=False),
        interpret=INTERPRET,
    )
    def spmm(hs_hbm, idx_hbm, out_hbm, ib, gb, zt, acc, gsem, ssem):
        cid = lax.axis_index("c")
        sid = lax.axis_index("s")
        wid = sid * NSC + cid

        zeros = jnp.zeros((16,), jnp.float32)
        def zbody(i, _):
            for kk in range(d // 16):
                zt[i, pl.ds(kk * 16, 16)] = zeros
            return 0
        lax.fori_loop(0, 16, zbody, 0)

        def zcopy(zb, _):
            dst = pl.multiple_of(sid * RPT + zb * 16, 8)
            pltpu.sync_copy(zt, acc.at[pl.ds(dst, 16)])
            return 0
        lax.fori_loop(0, RPT // 16, zcopy, 0)
        plsc.subcore_barrier()

        cbase = wid * k_chunks

        def ring_body(p, _):
            ci = cbase + p * NBUF
            for b in range(NBUF):
                pltpu.sync_copy(idx_hbm.at[ci + b], ib[b])
            cps = [pltpu.async_copy(hs_hbm.at[ib[b].at[0]], gb[b], gsem[b])
                   for b in range(NBUF)]
            scs = []
            for b in range(NBUF):
                cps[b].wait()
                scs.append(pltpu.async_copy(gb[b], acc.at[ib[b].at[1]],
                                            ssem[b], add=True))
            for b in range(NBUF):
                scs[b].wait()
            return 0

        lax.fori_loop(0, k_chunks // NBUF, ring_body, 0)
        plsc.subcore_barrier()

        src = acc.at[pl.ds(sid * RPT, RPT)]
        pltpu.sync_copy(src, out_hbm.at[cid].at[pl.ds(sid * RPT, RPT)])

    return spmm(hs, idxp)


def _pad_adj(rows, cols):
    e = rows.shape[0]
    rows = rows.astype(jnp.int32)
    cols = cols.astype(jnp.int32)
    grain = NW * EC * NBUF
    ep = -(-e // grain) * grain
    colsp = jnp.pad(cols, (0, ep - e))
    rowsp = jnp.pad(rows, (0, ep - e), constant_values=NWR - 8)
    idx = jnp.stack([colsp.reshape(-1, EC), rowsp.reshape(-1, EC)], axis=1)
    k_chunks = ep // (NW * EC)
    return idx, k_chunks


def _deg_sc_call(k1, idx1, k2, idx2):
    mesh = plsc.VectorSubcoreMesh(core_axis_name="c", subcore_axis_name="s",
                                  num_cores=2, num_subcores=16)

    @functools.partial(
        pl.kernel,
        out_type=jax.ShapeDtypeStruct((NSC, 2, NWR, 16), jnp.float32),
        mesh=mesh,
        scratch_types=[
            pltpu.VMEM((2, EC), jnp.int32),
            pltpu.VMEM((2, EC), jnp.int32),
            pltpu.VMEM((EC, 16), jnp.float32),
            pltpu.VMEM((64, 16), jnp.float32),
            pltpu.VMEM_SHARED((NWR, 16), jnp.float32),
            pltpu.VMEM_SHARED((NWR, 16), jnp.float32),
        ],
        compiler_params=pltpu.CompilerParams(use_tc_tiling_on_sc=False),
        interpret=INTERPRET,
    )
    def degk(idx1_hbm, idx2_hbm, out_hbm, rb0, rb1, ones_v, zt, acc1, acc2):
        cid = lax.axis_index("c")
        sid = lax.axis_index("s")
        wid = sid * NSC + cid

        zeros = jnp.zeros((16,), jnp.float32)
        ones = jnp.ones((16,), jnp.float32)

        def fill_body(i, _):
            zt[i, pl.ds(0, 16)] = zeros
            return 0
        lax.fori_loop(0, 64, fill_body, 0)

        def ones_body(i, _):
            ones_v[i, pl.ds(0, 16)] = ones
            return 0
        lax.fori_loop(0, EC, ones_body, 0)

        for zb in range(RPT // 64):
            pltpu.sync_copy(zt, acc1.at[pl.ds(sid * RPT + zb * 64, 64)])
            pltpu.sync_copy(zt, acc2.at[pl.ds(sid * RPT + zb * 64, 64)])
        plsc.subcore_barrier()

        def mk_body(idx_hbm, acc, k_chunks):
            cbase = wid * k_chunks

            def chunk_pair(p, _):
                ci = cbase + p * 2
                pltpu.sync_copy(idx_hbm.at[ci], rb0)
                pltpu.sync_copy(idx_hbm.at[ci + 1], rb1)
                pltpu.sync_copy(ones_v, acc.at[rb0.at[1]], add=True)
                pltpu.sync_copy(ones_v, acc.at[rb1.at[1]], add=True)
                return 0

            lax.fori_loop(0, k_chunks // 2, chunk_pair, 0)

        mk_body(idx1_hbm, acc1, k1)
        mk_body(idx2_hbm, acc2, k2)
        plsc.subcore_barrier()

        sl = pl.ds(sid * RPT, RPT)
        pltpu.sync_copy(acc1.at[sl], out_hbm.at[cid].at[0].at[sl])
        pltpu.sync_copy(acc2.at[sl], out_hbm.at[cid].at[1].at[sl])

    return degk(idx1, idx2)


def kernel(x, edge_index, W_fe, b_fe, bn_gamma, bn_beta, W_fp, b_fp,
           adj1_row, adj1_col, adj1_val, adj2_row, adj2_col, adj2_val):
    n = x.shape[0]
    i1, k1 = _pad_adj(adj1_row, adj1_col)
    i2, k2 = _pad_adj(adj2_row, adj2_col)

    degp = _deg_sc_call(k1, i1, k2, i2)
    deg1 = degp[0, 0, :, 0] + degp[1, 0, :, 0]
    deg2 = degp[0, 1, :, 0] + degp[1, 1, :, 0]
    mask = jnp.arange(NWR) < n
    d1p = jnp.where(mask & (deg1 > 0), lax.rsqrt(deg1), 0.0)
    d2p = jnp.where(mask & (deg2 > 0), lax.rsqrt(deg2), 0.0)

    xp = jnp.pad(x, ((0, NWR - n), (0, 0)))

    h, hs1, hs2 = _encoder(xp, W_fe, b_fe, d1p, d2p)

    pa = _spmm_sc_call(64, k1, hs1, i1)
    pb = _spmm_sc_call(64, k2, hs2, i2)
    h1, h1s1, h1s2 = _h1_producer(pa[0], pa[1], pb[0], pb[1],
                                  d1p, d2p, bn_gamma, bn_beta, NWR)

    qa = _spmm_sc_call(128, k1, h1s1, i1)
    qb = _spmm_sc_call(128, k2, h1s2, i2)

    out = _final(h, h1, qa[0], qa[1], qb[0], qb[1],
                 d1p, d2p, W_fp, b_fp, NWR)
    return out[:n]
